# baseline (speedup 1.0000x reference)
import functools

import jax
import jax.numpy as jnp
from jax import lax
from jax.experimental import pallas as pl
from jax.experimental.pallas import tpu as pltpu
from jax.experimental.pallas import tpu_sc as plsc

VOC = 1000000
NB_FULL = 7812
R_ROWS = 500000

NC, NS = 2, 16
NW = NC * NS

_mesh = plsc.VectorSubcoreMesh(
    core_axis_name="c", subcore_axis_name="s", num_cores=NC, num_subcores=NS)


def _splat(x):
  return jnp.zeros((16,), jnp.int32) + x


@functools.partial(
    pl.kernel,
    out_type=jax.ShapeDtypeStruct((R_ROWS, 128), jnp.float32),
    mesh=_mesh,
    compiler_params=pltpu.CompilerParams(
        use_tc_tiling_on_sc=True, needs_layout_passes=False),
    scratch_types=[
        pltpu.VMEM((64, 128), jnp.float32),
        pltpu.VMEM((64, 128), jnp.float32),
        pltpu.VMEM((64, 64), jnp.float32),
        pltpu.VMEM((32, 128), jnp.float32),
    ],
)
def _k1(tableT, r_out, in_v, out_v, tin_v, tout_v):
  w = lax.axis_index("s") * NC + lax.axis_index("c")
  iota = lax.iota(jnp.int32, 16)

  def transpose_rows(src, dst, r0, nrows):
    @pl.loop(0, nrows)
    def _(r):
      for half in range(2):
        lvec = _splat(2 * (r0 + r) + half)
        for c0 in range(0, 64, 16):
          v = plsc.load_gather(src, [iota + c0, lvec])
          dst[r, pl.ds(half * 64 + c0, 16)] = v

  nper = (NB_FULL + NW - 1) // NW

  @pl.loop(0, nper)
  def _(k):
    j = w + NW * k

    @pl.when(j < NB_FULL)
    def _():
      pltpu.sync_copy(tableT.at[:, pl.ds(j * 128, 128)], in_v)
      transpose_rows(in_v, out_v, 0, 64)
      pltpu.sync_copy(out_v, r_out.at[pl.ds(j * 64, 64)])

  @pl.when(w == 0)
  def _():
    pltpu.sync_copy(tableT.at[:, pl.ds(NB_FULL * 128, 64)], tin_v)
    transpose_rows(tin_v, tout_v, 0, 32)
    pltpu.sync_copy(tout_v, r_out.at[pl.ds(NB_FULL * 64, 32)])


@functools.partial(
    pl.kernel,
    out_type=jax.ShapeDtypeStruct((200, 64, 4096), jnp.float32),
    mesh=_mesh,
    compiler_params=pltpu.CompilerParams(
        use_tc_tiling_on_sc=True, needs_layout_passes=False),
    scratch_types=[
        pltpu.VMEM((200, 128), jnp.int32),
        pltpu.VMEM((200, 128), jnp.int32),
        pltpu.VMEM((128, 128), jnp.float32),
        pltpu.VMEM((128, 128), jnp.float32),
        pltpu.VMEM((64, 128), jnp.float32),
        pltpu.VMEM((64, 128), jnp.float32),
        pltpu.SemaphoreType.DMA,
        pltpu.SemaphoreType.DMA,
        pltpu.SemaphoreType.DMA,
        pltpu.SemaphoreType.DMA,
    ],
)
def _k2(idxT, r_in, out, idx_v, idx2_v, g0, g1, o0, o1, gs0, gs1, os0, os1):
  w = lax.axis_index("s") * NC + lax.axis_index("c")
  iota = lax.iota(jnp.int32, 16)

  pltpu.sync_copy(idxT.at[:, pl.ds(w * 128, 128)], idx_v)

  @pl.loop(0, 200)
  def _(s):
    for c0 in range(0, 128, 16):
      v = idx_v[s, pl.ds(c0, 16)]
      idx2_v[s, pl.ds(c0, 16)] = lax.shift_right_logical(v, 1)

  gbufs = (g0, g1)
  gsems = (gs0, gs1)
  obufs = (o0, o1)
  osems = (os0, os1)

  def start_gather(s, b):
    pltpu.async_copy(r_in.at[idx2_v.at[s]], gbufs[b], gsems[b])

  def wait_gather(b):
    pltpu.make_async_copy(
        r_in.at[pl.ds(0, 128)], gbufs[b], gsems[b]).wait()

  def transpose_block(s, b):
    g = gbufs[b]
    obuf = obufs[b]
    for b0 in range(0, 128, 16):
      lsb64 = (idx_v[s, pl.ds(b0, 16)] & 1) * 64

      @pl.loop(0, 64)
      def _(f):
        v = plsc.load_gather(g, [iota + b0, lsb64 + f])
        obuf[f, pl.ds(b0, 16)] = v

  def start_out(s, b):
    pltpu.async_copy(
        obufs[b], out.at[s, :, pl.ds(w * 128, 128)], osems[b])

  def wait_out(b):
    pltpu.make_async_copy(
        obufs[b], out.at[0, :, pl.ds(w * 128, 128)], osems[b]).wait()

  start_gather(0, 0)

  @pl.loop(0, 200, step=2)
  def _(s):
    start_gather(s + 1, 1)
    wait_gather(0)

    @pl.when(s >= 2)
    def _():
      wait_out(0)

    transpose_block(s, 0)
    start_out(s, 0)

    @pl.when(s + 2 < 200)
    def _():
      start_gather(s + 2, 0)

    wait_gather(1)

    @pl.when(s >= 2)
    def _():
      wait_out(1)

    transpose_block(s + 1, 1)
    start_out(s + 1, 1)

  wait_out(0)
  wait_out(1)


def kernel(word_indices, table):
  r = _k1(table.T)
  outT = _k2(word_indices.T, r)
  return outT.transpose(2, 0, 1)


# speedup vs baseline: 1.1155x; 1.1155x over previous
import functools

import jax
import jax.numpy as jnp
from jax import lax
from jax.experimental import pallas as pl
from jax.experimental.pallas import tpu as pltpu
from jax.experimental.pallas import tpu_sc as plsc

VOC = 1000000
NB_FULL = 7812
R_ROWS = 500000
BPW = 245

NC, NS = 2, 16
NW = NC * NS

_mesh = plsc.VectorSubcoreMesh(
    core_axis_name="c", subcore_axis_name="s", num_cores=NC, num_subcores=NS)

_params = pltpu.CompilerParams(
    use_tc_tiling_on_sc=True, needs_layout_passes=False)


def _splat(x):
  return jnp.zeros((16,), jnp.int32) + x


@functools.partial(
    pl.kernel,
    out_type=jax.ShapeDtypeStruct((R_ROWS, 128), jnp.float32),
    mesh=_mesh,
    compiler_params=_params,
    scratch_types=[
        pltpu.VMEM((2, 64, 128), jnp.float32),
        pltpu.VMEM((2, 64, 128), jnp.float32),
        pltpu.VMEM((64, 64), jnp.float32),
        pltpu.VMEM((32, 128), jnp.float32),
        pltpu.SemaphoreType.DMA,
        pltpu.SemaphoreType.DMA,
        pltpu.SemaphoreType.DMA,
        pltpu.SemaphoreType.DMA,
    ],
)
def _k1(tableT, r_out, in_v, out_v, tin_v, tout_v, is0, is1, os0, os1):
  w = lax.axis_index("s") * NC + lax.axis_index("c")
  iota = lax.iota(jnp.int32, 16)
  isems = (is0, is1)
  osems = (os0, os1)
  j0 = w * BPW

  def start_in(j, b):
    pltpu.async_copy(tableT.at[:, pl.ds(j * 128, 128)], in_v.at[b], isems[b])

  def wait_in(b):
    pltpu.make_async_copy(
        tableT.at[:, pl.ds(0, 128)], in_v.at[b], isems[b]).wait()

  def start_out(j, b):
    pltpu.async_copy(out_v.at[b], r_out.at[pl.ds(j * 64, 64)], osems[b])

  def wait_out(b):
    pltpu.make_async_copy(
        out_v.at[b], r_out.at[pl.ds(0, 64)], osems[b]).wait()

  def transpose_block(src, dst, nrows):
    @pl.loop(0, nrows, unroll=4)
    def _(r):
      for half in range(2):
        lvec = _splat(2 * r + half)
        for c0 in range(0, 64, 16):
          v = plsc.load_gather(src, [iota + c0, lvec])
          dst[r, pl.ds(half * 64 + c0, 16)] = v

  @pl.when(j0 < NB_FULL)
  def _():
    start_in(j0, 0)

  @pl.when(j0 + 1 < NB_FULL)
  def _():
    start_in(j0 + 1, 1)

  @pl.loop(0, BPW + 1, step=2)
  def _(k):
    for half in range(2):
      kk = k + half
      j = j0 + kk
      b = half

      @pl.when((j < NB_FULL) & (kk < BPW))
      def _():
        wait_in(b)

        @pl.when(kk >= 2)
        def _():
          wait_out(b)

        transpose_block(in_v.at[b], out_v.at[b], 64)
        start_out(j, b)

        @pl.when((j + 2 < NB_FULL) & (kk + 2 < BPW))
        def _():
          start_in(j + 2, b)

  @pl.when(j0 < NB_FULL)
  def _():
    wait_out(0)

  @pl.when(j0 + 1 < NB_FULL)
  def _():
    wait_out(1)

  @pl.when(w == 0)
  def _():
    pltpu.sync_copy(tableT.at[:, pl.ds(NB_FULL * 128, 64)], tin_v)

    @pl.loop(0, 32, unroll=4)
    def _(r):
      for half in range(2):
        lvec = _splat(2 * r + half)
        for c0 in range(0, 64, 16):
          v = plsc.load_gather(tin_v, [iota + c0, lvec])
          tout_v[r, pl.ds(half * 64 + c0, 16)] = v

    pltpu.sync_copy(tout_v, r_out.at[pl.ds(NB_FULL * 64, 32)])


@functools.partial(
    pl.kernel,
    out_type=jax.ShapeDtypeStruct((200, 64, 4096), jnp.float32),
    mesh=_mesh,
    compiler_params=_params,
    scratch_types=[
        pltpu.VMEM((200, 128), jnp.int32),
        pltpu.VMEM((200, 128), jnp.int32),
        pltpu.VMEM((128, 128), jnp.float32),
        pltpu.VMEM((128, 128), jnp.float32),
        pltpu.VMEM((64, 128), jnp.float32),
        pltpu.VMEM((64, 128), jnp.float32),
        pltpu.SemaphoreType.DMA,
        pltpu.SemaphoreType.DMA,
        pltpu.SemaphoreType.DMA,
        pltpu.SemaphoreType.DMA,
    ],
)
def _k2(idxT, r_in, out, idx_v, idx2_v, g0, g1, o0, o1, gs0, gs1, os0, os1):
  w = lax.axis_index("s") * NC + lax.axis_index("c")
  iota = lax.iota(jnp.int32, 16)

  pltpu.sync_copy(idxT.at[:, pl.ds(w * 128, 128)], idx_v)

  @pl.loop(0, 200, unroll=2)
  def _(s):
    for c0 in range(0, 128, 16):
      v = idx_v[s, pl.ds(c0, 16)]
      idx2_v[s, pl.ds(c0, 16)] = lax.shift_right_logical(v, 1)

  gbufs = (g0, g1)
  gsems = (gs0, gs1)
  obufs = (o0, o1)
  osems = (os0, os1)

  def start_gather(s, b):
    pltpu.async_copy(r_in.at[idx2_v.at[s]], gbufs[b], gsems[b])

  def wait_gather(b):
    pltpu.make_async_copy(r_in.at[pl.ds(0, 128)], gbufs[b], gsems[b]).wait()

  def transpose_block(s, b):
    g = gbufs[b]
    obuf = obufs[b]
    for b0 in range(0, 128, 16):
      lsb64 = (idx_v[s, pl.ds(b0, 16)] & 1) * 64

      @pl.loop(0, 64, unroll=8)
      def _(f):
        v = plsc.load_gather(g, [iota + b0, lsb64 + f])
        obuf[f, pl.ds(b0, 16)] = v

  def start_out(s, b):
    pltpu.async_copy(obufs[b], out.at[s, :, pl.ds(w * 128, 128)], osems[b])

  def wait_out(b):
    pltpu.make_async_copy(
        obufs[b], out.at[0, :, pl.ds(w * 128, 128)], osems[b]).wait()

  start_gather(0, 0)

  @pl.loop(0, 200, step=2)
  def _(s):
    start_gather(s + 1, 1)
    wait_gather(0)

    @pl.when(s >= 2)
    def _():
      wait_out(0)

    transpose_block(s, 0)
    start_out(s, 0)

    @pl.when(s + 2 < 200)
    def _():
      start_gather(s + 2, 0)

    wait_gather(1)

    @pl.when(s >= 2)
    def _():
      wait_out(1)

    transpose_block(s + 1, 1)
    start_out(s + 1, 1)

  wait_out(0)
  wait_out(1)


def kernel(word_indices, table):
  r = _k1(table.T)
  outT = _k2(word_indices.T, r)
  return outT.transpose(2, 0, 1)


# speedup vs baseline: 2.0962x; 1.8791x over previous
import functools

import jax
import jax.numpy as jnp
from jax import lax
from jax.experimental import pallas as pl
from jax.experimental.pallas import tpu as pltpu
from jax.experimental.pallas import tpu_sc as plsc

VOC = 1000000
NB_FULL = 7812
R_ROWS = 500000
BPW = 245

NC, NS = 2, 16
NW = NC * NS

_mesh = plsc.VectorSubcoreMesh(
    core_axis_name="c", subcore_axis_name="s", num_cores=NC, num_subcores=NS)

_params = pltpu.CompilerParams(
    use_tc_tiling_on_sc=True, needs_layout_passes=False)


def _splat(x):
  return jnp.zeros((16,), jnp.int32) + x


@functools.partial(
    pl.kernel,
    out_type=jax.ShapeDtypeStruct((R_ROWS, 128), jnp.float32),
    mesh=_mesh,
    compiler_params=_params,
    scratch_types=[
        pltpu.VMEM((2, 64, 128), jnp.float32),
        pltpu.VMEM((2, 64, 128), jnp.float32),
        pltpu.VMEM((64, 64), jnp.float32),
        pltpu.VMEM((32, 128), jnp.float32),
        pltpu.SemaphoreType.DMA,
        pltpu.SemaphoreType.DMA,
        pltpu.SemaphoreType.DMA,
        pltpu.SemaphoreType.DMA,
    ],
)
def _k1(tableT, r_out, in_v, out_v, tin_v, tout_v, is0, is1, os0, os1):
  w = lax.axis_index("s") * NC + lax.axis_index("c")
  iota = lax.iota(jnp.int32, 16)
  isems = (is0, is1)
  osems = (os0, os1)
  j0 = w * BPW

  def start_in(j, b):
    pltpu.async_copy(tableT.at[:, pl.ds(j * 128, 128)], in_v.at[b], isems[b])

  def wait_in(b):
    pltpu.make_async_copy(
        tableT.at[:, pl.ds(0, 128)], in_v.at[b], isems[b]).wait()

  def start_out(j, b):
    pltpu.async_copy(out_v.at[b], r_out.at[pl.ds(j * 64, 64)], osems[b])

  def wait_out(b):
    pltpu.make_async_copy(
        out_v.at[b], r_out.at[pl.ds(0, 64)], osems[b]).wait()

  def transpose_block(src, dst, nrows):
    @plsc.parallel_loop(0, nrows, unroll=4)
    def _(r):
      for half in range(2):
        lvec = _splat(2 * r + half)
        for c0 in range(0, 64, 16):
          v = plsc.load_gather(src, [iota + c0, lvec])
          dst[r, pl.ds(half * 64 + c0, 16)] = v

  @pl.when(j0 < NB_FULL)
  def _():
    start_in(j0, 0)

  @pl.when(j0 + 1 < NB_FULL)
  def _():
    start_in(j0 + 1, 1)

  @pl.loop(0, BPW + 1, step=2)
  def _(k):
    for half in range(2):
      kk = k + half
      j = j0 + kk
      b = half

      @pl.when((j < NB_FULL) & (kk < BPW))
      def _():
        wait_in(b)

        @pl.when(kk >= 2)
        def _():
          wait_out(b)

        transpose_block(in_v.at[b], out_v.at[b], 64)
        start_out(j, b)

        @pl.when((j + 2 < NB_FULL) & (kk + 2 < BPW))
        def _():
          start_in(j + 2, b)

  @pl.when(j0 < NB_FULL)
  def _():
    wait_out(0)

  @pl.when(j0 + 1 < NB_FULL)
  def _():
    wait_out(1)

  @pl.when(w == 0)
  def _():
    pltpu.sync_copy(tableT.at[:, pl.ds(NB_FULL * 128, 64)], tin_v)

    @plsc.parallel_loop(0, 32, unroll=4)
    def _(r):
      for half in range(2):
        lvec = _splat(2 * r + half)
        for c0 in range(0, 64, 16):
          v = plsc.load_gather(tin_v, [iota + c0, lvec])
          tout_v[r, pl.ds(half * 64 + c0, 16)] = v

    pltpu.sync_copy(tout_v, r_out.at[pl.ds(NB_FULL * 64, 32)])


@functools.partial(
    pl.kernel,
    out_type=jax.ShapeDtypeStruct((200, 64, 4096), jnp.float32),
    mesh=_mesh,
    compiler_params=_params,
    scratch_types=[
        pltpu.VMEM((200, 128), jnp.int32),
        pltpu.VMEM((200, 128), jnp.int32),
        pltpu.VMEM((128, 128), jnp.float32),
        pltpu.VMEM((128, 128), jnp.float32),
        pltpu.VMEM((64, 128), jnp.float32),
        pltpu.VMEM((64, 128), jnp.float32),
        pltpu.SemaphoreType.DMA,
        pltpu.SemaphoreType.DMA,
        pltpu.SemaphoreType.DMA,
        pltpu.SemaphoreType.DMA,
    ],
)
def _k2(idxT, r_in, out, idx_v, idx2_v, g0, g1, o0, o1, gs0, gs1, os0, os1):
  w = lax.axis_index("s") * NC + lax.axis_index("c")
  iota = lax.iota(jnp.int32, 16)

  pltpu.sync_copy(idxT.at[:, pl.ds(w * 128, 128)], idx_v)

  @plsc.parallel_loop(0, 200, unroll=2)
  def _(s):
    for c0 in range(0, 128, 16):
      v = idx_v[s, pl.ds(c0, 16)]
      idx2_v[s, pl.ds(c0, 16)] = lax.shift_right_logical(v, 1)

  gbufs = (g0, g1)
  gsems = (gs0, gs1)
  obufs = (o0, o1)
  osems = (os0, os1)

  def start_gather(s, b):
    pltpu.async_copy(r_in.at[idx2_v.at[s]], gbufs[b], gsems[b])

  def wait_gather(b):
    pltpu.make_async_copy(r_in.at[pl.ds(0, 128)], gbufs[b], gsems[b]).wait()

  def transpose_block(s, b):
    g = gbufs[b]
    obuf = obufs[b]
    for b0 in range(0, 128, 16):
      lsb64 = (idx_v[s, pl.ds(b0, 16)] & 1) * 64

      @plsc.parallel_loop(0, 64, unroll=8)
      def _(f):
        v = plsc.load_gather(g, [iota + b0, lsb64 + f])
        obuf[f, pl.ds(b0, 16)] = v

  def start_out(s, b):
    pltpu.async_copy(obufs[b], out.at[s, :, pl.ds(w * 128, 128)], osems[b])

  def wait_out(b):
    pltpu.make_async_copy(
        obufs[b], out.at[0, :, pl.ds(w * 128, 128)], osems[b]).wait()

  start_gather(0, 0)

  @pl.loop(0, 200, step=2)
  def _(s):
    start_gather(s + 1, 1)
    wait_gather(0)

    @pl.when(s >= 2)
    def _():
      wait_out(0)

    transpose_block(s, 0)
    start_out(s, 0)

    @pl.when(s + 2 < 200)
    def _():
      start_gather(s + 2, 0)

    wait_gather(1)

    @pl.when(s >= 2)
    def _():
      wait_out(1)

    transpose_block(s + 1, 1)
    start_out(s + 1, 1)

  wait_out(0)
  wait_out(1)


def kernel(word_indices, table):
  r = _k1(table.T)
  outT = _k2(word_indices.T, r)
  return outT.transpose(2, 0, 1)
